# baseline (speedup 1.0000x reference)
import functools

import jax
import jax.numpy as jnp
import numpy as np
from jax import lax
from jax.experimental import pallas as pl
from jax.experimental.pallas import tpu as pltpu


def _conv_stats_kernel(m_ref, x_ref, w1_ref, w2_ref, w3_ref, b_ref, cmask_ref,
                       z_ref, mout_ref, s1_ref, s2_ref,
                       m_scr, p_scr,
                       *, H, W, Wp, L2, LP, LQ, Lm):
    f32 = jnp.float32
    bf16 = jnp.bfloat16
    Cin = x_ref.shape[1]

    m_scr[...] = jnp.zeros((1, Lm), f32)
    for h in range(H):
        m_scr[:, (h + 2) * Wp + 2:(h + 2) * Wp + 2 + W] = m_ref[0, :, h, :]

    mslab = m_scr[...]
    msum = jnp.zeros((1, LP), f32)
    for kh in range(3):
        for kw in range(3):
            s = kh * Wp + kw
            msum = msum + mslab[:, s:s + LP]
    m1 = jnp.where(msum == 0.0, 1.0, msum)

    p_scr[...] = jnp.zeros((Cin, LP), bf16)
    for h in range(H):
        o = (h + 1) * Wp + 1
        xrow = x_ref[0, :, h, :].astype(f32)
        p_scr[:, o:o + W] = (m1[:, o:o + W] * xrow).astype(bf16)
    P = p_scr[...]

    Q1 = jnp.concatenate([P[:, :LQ], P[:, 1:LQ + 1]], axis=0)
    QW = jnp.concatenate([P[:, :LQ], P[:, Wp:Wp + LQ]], axis=0)
    acc = jnp.dot(w1_ref[0], Q1[:, 0:L2], preferred_element_type=f32)
    acc = acc + jnp.dot(w1_ref[1], Q1[:, Wp:Wp + L2], preferred_element_type=f32)
    acc = acc + jnp.dot(w1_ref[2], Q1[:, 2 * Wp:2 * Wp + L2], preferred_element_type=f32)
    acc = acc + jnp.dot(w2_ref[...], QW[:, 2:2 + L2], preferred_element_type=f32)
    acc = acc + jnp.dot(w3_ref[...], P[:, 2 * Wp + 2:2 * Wp + 2 + L2],
                        preferred_element_type=f32)
    y = acc + b_ref[...]

    off = Wp + 1
    inv_m = 1.0 / m1[:, off:off + L2]
    z = y * inv_m

    z_ref[0] = z.astype(bf16)
    mout_ref[0] = msum[:, off:off + L2]

    zm = z * cmask_ref[...]
    s1_ref[0] = jnp.sum(zm, axis=1, keepdims=True)
    s2_ref[0] = jnp.sum(zm * z, axis=1, keepdims=True)


def _bn_relu_kernel(z_ref, mo_ref, a_ref, b_ref, out_ref, mout_ref, *, H, W, Wp):
    zv = z_ref[0][:, :, :W].astype(jnp.float32)
    out_ref[0] = jnp.maximum(zv * a_ref[...] + b_ref[...], 0.0)
    mo = mo_ref[0]
    for h in range(H):
        mout_ref[0, :, h, :] = mo[:, h * Wp:h * Wp + W]


def kernel(x, M, w_I, b_I, gamma, beta):
    N, Cin, H, W = x.shape
    Cout = w_I.shape[0]
    eps = 1e-5
    f32 = jnp.float32
    bf16 = jnp.bfloat16

    Wp = W + 2
    L2 = H * Wp
    LP = (H + 5) * Wp
    LQ = (H + 3) * Wp
    Lm = 2 * Wp + 3 + LP

    w_tap = w_I.astype(f32).transpose(2, 3, 0, 1).reshape(9, Cout, Cin)
    w1 = jnp.concatenate([w_tap[0::3], w_tap[1::3]], axis=2).astype(bf16)
    w2 = jnp.concatenate([w_tap[2], w_tap[5]], axis=1).astype(bf16)
    w3 = w_tap[8].astype(bf16)
    bias = b_I.astype(f32).reshape(Cout, 1)

    idx = np.arange(L2)
    cmask = jnp.asarray((idx % Wp < W).astype(np.float32)).reshape(1, L2)

    cparams = pltpu.CompilerParams(
        dimension_semantics=("parallel",),
        vmem_limit_bytes=64 * 1024 * 1024,
    )

    kern1 = functools.partial(_conv_stats_kernel, H=H, W=W, Wp=Wp,
                              L2=L2, LP=LP, LQ=LQ, Lm=Lm)
    z, mo_flat, ssum, ssq = pl.pallas_call(
        kern1,
        grid=(N,),
        in_specs=[
            pl.BlockSpec((1, 1, H, W), lambda g: (g, 0, 0, 0)),
            pl.BlockSpec((1, Cin, H, W), lambda g: (g, 0, 0, 0)),
            pl.BlockSpec((3, Cout, 2 * Cin), lambda g: (0, 0, 0)),
            pl.BlockSpec((Cout, 2 * Cin), lambda g: (0, 0)),
            pl.BlockSpec((Cout, Cin), lambda g: (0, 0)),
            pl.BlockSpec((Cout, 1), lambda g: (0, 0)),
            pl.BlockSpec((1, L2), lambda g: (0, 0)),
        ],
        out_specs=(
            pl.BlockSpec((1, Cout, L2), lambda g: (g, 0, 0)),
            pl.BlockSpec((1, 1, L2), lambda g: (g, 0, 0)),
            pl.BlockSpec((1, Cout, 1), lambda g: (g, 0, 0)),
            pl.BlockSpec((1, Cout, 1), lambda g: (g, 0, 0)),
        ),
        out_shape=(
            jax.ShapeDtypeStruct((N, Cout, L2), bf16),
            jax.ShapeDtypeStruct((N, 1, L2), f32),
            jax.ShapeDtypeStruct((N, Cout, 1), f32),
            jax.ShapeDtypeStruct((N, Cout, 1), f32),
        ),
        scratch_shapes=[
            pltpu.VMEM((1, Lm), f32),
            pltpu.VMEM((Cin, LP), bf16),
        ],
        compiler_params=cparams,
    )(M, x, w1, w2, w3, bias, cmask)

    cnt = float(N * H * W)
    mean = jnp.sum(ssum, axis=0)[:, 0] / cnt
    var = jnp.maximum(jnp.sum(ssq, axis=0)[:, 0] / cnt - mean * mean, 0.0)
    a = gamma.astype(f32) * lax.rsqrt(var + eps)
    bshift = beta.astype(f32) - a * mean
    a = a.reshape(Cout, 1, 1)
    bshift = bshift.reshape(Cout, 1, 1)

    z4 = z.reshape(N, Cout, H, Wp)

    kern2 = functools.partial(_bn_relu_kernel, H=H, W=W, Wp=Wp)
    x_out, m_out = pl.pallas_call(
        kern2,
        grid=(N,),
        in_specs=[
            pl.BlockSpec((1, Cout, H, Wp), lambda g: (g, 0, 0, 0)),
            pl.BlockSpec((1, 1, L2), lambda g: (g, 0, 0)),
            pl.BlockSpec((Cout, 1, 1), lambda g: (0, 0, 0)),
            pl.BlockSpec((Cout, 1, 1), lambda g: (0, 0, 0)),
        ],
        out_specs=(
            pl.BlockSpec((1, Cout, H, W), lambda g: (g, 0, 0, 0)),
            pl.BlockSpec((1, 1, H, W), lambda g: (g, 0, 0, 0)),
        ),
        out_shape=(
            jax.ShapeDtypeStruct((N, Cout, H, W), f32),
            jax.ShapeDtypeStruct((N, 1, H, W), f32),
        ),
        compiler_params=cparams,
    )(z4, mo_flat, a, bshift)

    return x_out, m_out


# speedup vs baseline: 1.2838x; 1.2838x over previous
import functools

import jax
import jax.numpy as jnp
import numpy as np
from jax import lax
from jax.experimental import pallas as pl
from jax.experimental.pallas import tpu as pltpu


def _conv_stats_kernel(m_ref, x_ref, w1_ref, w2_ref, w3_ref, b_ref, cmask_ref,
                       z_ref, mout_ref, s1_ref, s2_ref,
                       *, Wp, L2, LP, LQ):
    f32 = jnp.float32
    mslab = m_ref[0]
    msum = jnp.zeros((1, LP), f32)
    for kh in range(3):
        for kw in range(3):
            s = kh * Wp + kw
            msum = msum + mslab[:, s:s + LP]
    m1 = jnp.where(msum == 0.0, 1.0, msum)

    xs = x_ref[0]
    P = (m1 * xs.astype(f32)).astype(jnp.bfloat16)

    Q1 = jnp.concatenate([P[:, :LQ], P[:, 1:LQ + 1]], axis=0)
    QW = jnp.concatenate([P[:, :LQ], P[:, Wp:Wp + LQ]], axis=0)
    acc = jnp.dot(w1_ref[0], Q1[:, 0:L2], preferred_element_type=f32)
    acc = acc + jnp.dot(w1_ref[1], Q1[:, Wp:Wp + L2], preferred_element_type=f32)
    acc = acc + jnp.dot(w1_ref[2], Q1[:, 2 * Wp:2 * Wp + L2], preferred_element_type=f32)
    acc = acc + jnp.dot(w2_ref[...], QW[:, 2:2 + L2], preferred_element_type=f32)
    acc = acc + jnp.dot(w3_ref[...], P[:, 2 * Wp + 2:2 * Wp + 2 + L2],
                        preferred_element_type=f32)
    y = acc + b_ref[...]

    off = Wp + 1
    inv_m = 1.0 / m1[:, off:off + L2]
    z = y * inv_m

    z_ref[0] = z.astype(jnp.bfloat16)
    mout_ref[0] = msum[:, off:off + L2]

    zm = z * cmask_ref[...]
    s1_ref[0] = jnp.sum(zm, axis=1, keepdims=True)
    s2_ref[0] = jnp.sum(zm * z, axis=1, keepdims=True)


def _bn_relu_kernel(z_ref, a_ref, b_ref, o_ref):
    o_ref[0] = jnp.maximum(z_ref[0].astype(jnp.float32) * a_ref[...] + b_ref[...],
                           0.0)


def kernel(x, M, w_I, b_I, gamma, beta):
    N, Cin, H, W = x.shape
    Cout = w_I.shape[0]
    eps = 1e-5
    f32 = jnp.float32
    bf16 = jnp.bfloat16

    Wp = W + 2
    L2 = H * Wp
    LP = (H + 5) * Wp
    LQ = (H + 3) * Wp
    Lm = 2 * Wp + 3 + LP

    xb = x.astype(f32).astype(bf16)
    x_flat = jnp.pad(xb, ((0, 0), (0, 0), (1, 4), (1, 1))).reshape(N, Cin, LP)

    Mf = M.astype(f32)
    m_flat = jnp.pad(Mf, ((0, 0), (0, 0), (2, 2), (1, 1))).reshape(N, 1, (H + 4) * Wp)
    m_flat = jnp.pad(m_flat, ((0, 0), (0, 0), (1, Lm - 1 - (H + 4) * Wp)))

    w_tap = w_I.astype(f32).transpose(2, 3, 0, 1).reshape(9, Cout, Cin)
    w1 = jnp.concatenate([w_tap[0::3], w_tap[1::3]], axis=2).astype(bf16)
    w2 = jnp.concatenate([w_tap[2], w_tap[5]], axis=1).astype(bf16)
    w3 = w_tap[8].astype(bf16)
    bias = b_I.astype(f32).reshape(Cout, 1)

    idx = np.arange(L2)
    cmask = jnp.asarray((idx % Wp < W).astype(np.float32)).reshape(1, L2)

    cparams = pltpu.CompilerParams(
        dimension_semantics=("parallel",),
        vmem_limit_bytes=64 * 1024 * 1024,
    )

    kern1 = functools.partial(_conv_stats_kernel, Wp=Wp, L2=L2, LP=LP, LQ=LQ)
    z, mo_flat, ssum, ssq = pl.pallas_call(
        kern1,
        grid=(N,),
        in_specs=[
            pl.BlockSpec((1, 1, Lm), lambda g: (g, 0, 0)),
            pl.BlockSpec((1, Cin, LP), lambda g: (g, 0, 0)),
            pl.BlockSpec((3, Cout, 2 * Cin), lambda g: (0, 0, 0)),
            pl.BlockSpec((Cout, 2 * Cin), lambda g: (0, 0)),
            pl.BlockSpec((Cout, Cin), lambda g: (0, 0)),
            pl.BlockSpec((Cout, 1), lambda g: (0, 0)),
            pl.BlockSpec((1, L2), lambda g: (0, 0)),
        ],
        out_specs=(
            pl.BlockSpec((1, Cout, L2), lambda g: (g, 0, 0)),
            pl.BlockSpec((1, 1, L2), lambda g: (g, 0, 0)),
            pl.BlockSpec((1, Cout, 1), lambda g: (g, 0, 0)),
            pl.BlockSpec((1, Cout, 1), lambda g: (g, 0, 0)),
        ),
        out_shape=(
            jax.ShapeDtypeStruct((N, Cout, L2), bf16),
            jax.ShapeDtypeStruct((N, 1, L2), f32),
            jax.ShapeDtypeStruct((N, Cout, 1), f32),
            jax.ShapeDtypeStruct((N, Cout, 1), f32),
        ),
        compiler_params=cparams,
    )(m_flat, x_flat, w1, w2, w3, bias, cmask)

    cnt = float(N * H * W)
    mean = jnp.sum(ssum, axis=0)[:, 0] / cnt
    var = jnp.maximum(jnp.sum(ssq, axis=0)[:, 0] / cnt - mean * mean, 0.0)
    a = gamma.astype(f32) * lax.rsqrt(var + eps)
    bshift = beta.astype(f32) - a * mean
    a = a.reshape(Cout, 1)
    bshift = bshift.reshape(Cout, 1)

    out_flat = pl.pallas_call(
        _bn_relu_kernel,
        grid=(N,),
        in_specs=[
            pl.BlockSpec((1, Cout, L2), lambda g: (g, 0, 0)),
            pl.BlockSpec((Cout, 1), lambda g: (0, 0)),
            pl.BlockSpec((Cout, 1), lambda g: (0, 0)),
        ],
        out_specs=pl.BlockSpec((1, Cout, L2), lambda g: (g, 0, 0)),
        out_shape=jax.ShapeDtypeStruct((N, Cout, L2), f32),
        compiler_params=cparams,
    )(z, a, bshift)

    x_out = out_flat.reshape(N, Cout, H, Wp)[..., :W]
    m_out = mo_flat.reshape(N, 1, H, Wp)[..., :W]
    return x_out, m_out


# speedup vs baseline: 1.3653x; 1.0634x over previous
import functools

import jax
import jax.numpy as jnp
import numpy as np
from jax import lax
from jax.experimental import pallas as pl
from jax.experimental.pallas import tpu as pltpu


def _conv_stats_kernel(m_ref, x_ref, w1_ref, w2_ref, w3_ref, b_ref, cmask_ref,
                       z_ref, mout_ref, s1_ref, s2_ref,
                       *, Wp, L2, LP, LQ):
    f32 = jnp.float32
    mslab = m_ref[0]
    msum = jnp.zeros((1, LP), f32)
    for kh in range(3):
        for kw in range(3):
            s = kh * Wp + kw
            msum = msum + mslab[:, s:s + LP]
    m1 = jnp.where(msum == 0.0, 1.0, msum)

    xs = x_ref[0]
    P = m1.astype(jnp.bfloat16) * xs

    Q1 = jnp.concatenate([P[:, :LQ], P[:, 1:LQ + 1]], axis=0)
    QW = jnp.concatenate([P[:, :LQ], P[:, Wp:Wp + LQ]], axis=0)
    acc = jnp.dot(w1_ref[0], Q1[:, 0:L2], preferred_element_type=f32)
    acc = acc + jnp.dot(w1_ref[1], Q1[:, Wp:Wp + L2], preferred_element_type=f32)
    acc = acc + jnp.dot(w1_ref[2], Q1[:, 2 * Wp:2 * Wp + L2], preferred_element_type=f32)
    acc = acc + jnp.dot(w2_ref[...], QW[:, 2:2 + L2], preferred_element_type=f32)
    acc = acc + jnp.dot(w3_ref[...], P[:, 2 * Wp + 2:2 * Wp + 2 + L2],
                        preferred_element_type=f32)
    y = acc + b_ref[...]

    off = Wp + 1
    inv_m = 1.0 / m1[:, off:off + L2]
    z = y * inv_m

    z_ref[0] = z.astype(jnp.bfloat16)
    mout_ref[0] = msum[:, off:off + L2]

    zm = z * cmask_ref[...]
    s1_ref[0] = jnp.sum(zm, axis=1, keepdims=True)
    s2_ref[0] = jnp.sum(zm * z, axis=1, keepdims=True)


def _bn_relu_kernel(z_ref, a_ref, b_ref, o_ref):
    r = jnp.maximum(z_ref[0].astype(jnp.float32) * a_ref[...] + b_ref[...], 0.0)
    o_ref[0] = r.astype(jnp.bfloat16)


def kernel(x, M, w_I, b_I, gamma, beta):
    N, Cin, H, W = x.shape
    Cout = w_I.shape[0]
    eps = 1e-5
    f32 = jnp.float32
    bf16 = jnp.bfloat16

    Wp = W + 2
    L2 = H * Wp
    LP = (H + 5) * Wp
    LQ = (H + 3) * Wp
    Lm = 2 * Wp + 3 + LP

    xb = x.astype(f32).astype(bf16)
    x_flat = jnp.pad(xb, ((0, 0), (0, 0), (1, 4), (1, 1))).reshape(N, Cin, LP)

    Mf = M.astype(f32)
    m_flat = jnp.pad(Mf, ((0, 0), (0, 0), (2, 2), (1, 1))).reshape(N, 1, (H + 4) * Wp)
    m_flat = jnp.pad(m_flat, ((0, 0), (0, 0), (1, Lm - 1 - (H + 4) * Wp)))

    w_tap = w_I.astype(f32).transpose(2, 3, 0, 1).reshape(9, Cout, Cin)
    w1 = jnp.concatenate([w_tap[0::3], w_tap[1::3]], axis=2).astype(bf16)
    w2 = jnp.concatenate([w_tap[2], w_tap[5]], axis=1).astype(bf16)
    w3 = w_tap[8].astype(bf16)
    bias = b_I.astype(f32).reshape(Cout, 1)

    idx = np.arange(L2)
    cmask = jnp.asarray((idx % Wp < W).astype(np.float32)).reshape(1, L2)

    cparams = pltpu.CompilerParams(
        dimension_semantics=("parallel",),
        vmem_limit_bytes=64 * 1024 * 1024,
    )

    kern1 = functools.partial(_conv_stats_kernel, Wp=Wp, L2=L2, LP=LP, LQ=LQ)
    z, mo_flat, ssum, ssq = pl.pallas_call(
        kern1,
        grid=(N,),
        in_specs=[
            pl.BlockSpec((1, 1, Lm), lambda g: (g, 0, 0)),
            pl.BlockSpec((1, Cin, LP), lambda g: (g, 0, 0)),
            pl.BlockSpec((3, Cout, 2 * Cin), lambda g: (0, 0, 0)),
            pl.BlockSpec((Cout, 2 * Cin), lambda g: (0, 0)),
            pl.BlockSpec((Cout, Cin), lambda g: (0, 0)),
            pl.BlockSpec((Cout, 1), lambda g: (0, 0)),
            pl.BlockSpec((1, L2), lambda g: (0, 0)),
        ],
        out_specs=(
            pl.BlockSpec((1, Cout, L2), lambda g: (g, 0, 0)),
            pl.BlockSpec((1, 1, L2), lambda g: (g, 0, 0)),
            pl.BlockSpec((1, Cout, 1), lambda g: (g, 0, 0)),
            pl.BlockSpec((1, Cout, 1), lambda g: (g, 0, 0)),
        ),
        out_shape=(
            jax.ShapeDtypeStruct((N, Cout, L2), bf16),
            jax.ShapeDtypeStruct((N, 1, L2), f32),
            jax.ShapeDtypeStruct((N, Cout, 1), f32),
            jax.ShapeDtypeStruct((N, Cout, 1), f32),
        ),
        compiler_params=cparams,
    )(m_flat, x_flat, w1, w2, w3, bias, cmask)

    cnt = float(N * H * W)
    mean = jnp.sum(ssum, axis=0)[:, 0] / cnt
    var = jnp.maximum(jnp.sum(ssq, axis=0)[:, 0] / cnt - mean * mean, 0.0)
    a = gamma.astype(f32) * lax.rsqrt(var + eps)
    bshift = beta.astype(f32) - a * mean
    a = a.reshape(Cout, 1)
    bshift = bshift.reshape(Cout, 1)

    out_flat = pl.pallas_call(
        _bn_relu_kernel,
        grid=(N,),
        in_specs=[
            pl.BlockSpec((1, Cout, L2), lambda g: (g, 0, 0)),
            pl.BlockSpec((Cout, 1), lambda g: (0, 0)),
            pl.BlockSpec((Cout, 1), lambda g: (0, 0)),
        ],
        out_specs=pl.BlockSpec((1, Cout, L2), lambda g: (g, 0, 0)),
        out_shape=jax.ShapeDtypeStruct((N, Cout, L2), bf16),
        compiler_params=cparams,
    )(z, a, bshift)

    x_out = out_flat.reshape(N, Cout, H, Wp)[..., :W].astype(f32)
    m_out = mo_flat.reshape(N, 1, H, Wp)[..., :W]
    return x_out, m_out


# speedup vs baseline: 1.7045x; 1.2485x over previous
import functools

import jax
import jax.numpy as jnp
import numpy as np
from jax import lax
from jax.experimental import pallas as pl
from jax.experimental.pallas import tpu as pltpu


def _conv_stats_kernel(m_ref, x_ref, w1_ref, w2_ref, w3_ref, b_ref, cmask_ref,
                       z_ref, mout_ref, s1_ref, s2_ref,
                       m_scr, p_scr,
                       *, H, W, Wp, L2, LP, LQ, Lm):
    f32 = jnp.float32
    bf16 = jnp.bfloat16
    Cin = p_scr.shape[0]

    m_scr[...] = jnp.zeros((1, Lm), f32)
    for h in range(H):
        m_scr[:, (h + 2) * Wp + 2:(h + 2) * Wp + 2 + W] = m_ref[0][:, h * W:(h + 1) * W]

    mslab = m_scr[...]
    msum = jnp.zeros((1, LP), f32)
    for kh in range(3):
        for kw in range(3):
            s = kh * Wp + kw
            msum = msum + mslab[:, s:s + LP]
    m1 = jnp.where(msum == 0.0, 1.0, msum)
    m1b = m1.astype(bf16)

    p_scr[...] = jnp.zeros((Cin, LP), bf16)
    for h in range(H):
        o = (h + 1) * Wp + 1
        xrow = x_ref[0][:, h * W:(h + 1) * W].astype(bf16)
        p_scr[:, o:o + W] = m1b[:, o:o + W] * xrow
    P = p_scr[...]

    Q1 = jnp.concatenate([P[:, :LQ], P[:, 1:LQ + 1]], axis=0)
    QW = jnp.concatenate([P[:, :LQ], P[:, Wp:Wp + LQ]], axis=0)
    acc = jnp.dot(w1_ref[0], Q1[:, 0:L2], preferred_element_type=f32)
    acc = acc + jnp.dot(w1_ref[1], Q1[:, Wp:Wp + L2], preferred_element_type=f32)
    acc = acc + jnp.dot(w1_ref[2], Q1[:, 2 * Wp:2 * Wp + L2], preferred_element_type=f32)
    acc = acc + jnp.dot(w2_ref[...], QW[:, 2:2 + L2], preferred_element_type=f32)
    acc = acc + jnp.dot(w3_ref[...], P[:, 2 * Wp + 2:2 * Wp + 2 + L2],
                        preferred_element_type=f32)
    y = acc + b_ref[...]

    off = Wp + 1
    inv_m = 1.0 / m1[:, off:off + L2]
    z = y * inv_m

    z_ref[0] = z.astype(bf16)

    for h in range(H):
        mout_ref[0, :, h * W:(h + 1) * W] = msum[:, off + h * Wp:off + h * Wp + W]

    zm = z * cmask_ref[...]
    s1_ref[0] = jnp.sum(zm, axis=1, keepdims=True)
    s2_ref[0] = jnp.sum(zm * z, axis=1, keepdims=True)


def _bn_relu_kernel(z_ref, a_ref, b_ref, o_ref, *, H, W, Wp):
    r = jnp.maximum(z_ref[0].astype(jnp.float32) * a_ref[...] + b_ref[...], 0.0)
    for h in range(H):
        o_ref[0, :, h * W:(h + 1) * W] = r[:, h * Wp:h * Wp + W]


def kernel(x, M, w_I, b_I, gamma, beta):
    N, Cin, H, W = x.shape
    Cout = w_I.shape[0]
    eps = 1e-5
    f32 = jnp.float32
    bf16 = jnp.bfloat16

    Wp = W + 2
    L2 = H * Wp
    LP = (H + 5) * Wp
    LQ = (H + 3) * Wp
    Lm = 2 * Wp + 3 + LP
    HW = H * W

    xf = x.reshape(N, Cin, HW)
    mf = M.reshape(N, 1, HW)

    w_tap = w_I.astype(f32).transpose(2, 3, 0, 1).reshape(9, Cout, Cin)
    w1 = jnp.concatenate([w_tap[0::3], w_tap[1::3]], axis=2).astype(bf16)
    w2 = jnp.concatenate([w_tap[2], w_tap[5]], axis=1).astype(bf16)
    w3 = w_tap[8].astype(bf16)
    bias = b_I.astype(f32).reshape(Cout, 1)

    idx = np.arange(L2)
    cmask = jnp.asarray((idx % Wp < W).astype(np.float32)).reshape(1, L2)

    cparams = pltpu.CompilerParams(
        dimension_semantics=("parallel",),
        vmem_limit_bytes=64 * 1024 * 1024,
    )

    kern1 = functools.partial(_conv_stats_kernel, H=H, W=W, Wp=Wp,
                              L2=L2, LP=LP, LQ=LQ, Lm=Lm)
    z, mo_flat, ssum, ssq = pl.pallas_call(
        kern1,
        grid=(N,),
        in_specs=[
            pl.BlockSpec((1, 1, HW), lambda g: (g, 0, 0)),
            pl.BlockSpec((1, Cin, HW), lambda g: (g, 0, 0)),
            pl.BlockSpec((3, Cout, 2 * Cin), lambda g: (0, 0, 0)),
            pl.BlockSpec((Cout, 2 * Cin), lambda g: (0, 0)),
            pl.BlockSpec((Cout, Cin), lambda g: (0, 0)),
            pl.BlockSpec((Cout, 1), lambda g: (0, 0)),
            pl.BlockSpec((1, L2), lambda g: (0, 0)),
        ],
        out_specs=(
            pl.BlockSpec((1, Cout, L2), lambda g: (g, 0, 0)),
            pl.BlockSpec((1, 1, HW), lambda g: (g, 0, 0)),
            pl.BlockSpec((1, Cout, 1), lambda g: (g, 0, 0)),
            pl.BlockSpec((1, Cout, 1), lambda g: (g, 0, 0)),
        ),
        out_shape=(
            jax.ShapeDtypeStruct((N, Cout, L2), bf16),
            jax.ShapeDtypeStruct((N, 1, HW), f32),
            jax.ShapeDtypeStruct((N, Cout, 1), f32),
            jax.ShapeDtypeStruct((N, Cout, 1), f32),
        ),
        scratch_shapes=[
            pltpu.VMEM((1, Lm), f32),
            pltpu.VMEM((Cin, LP), bf16),
        ],
        compiler_params=cparams,
    )(mf, xf, w1, w2, w3, bias, cmask)

    cnt = float(N * H * W)
    mean = jnp.sum(ssum, axis=0)[:, 0] / cnt
    var = jnp.maximum(jnp.sum(ssq, axis=0)[:, 0] / cnt - mean * mean, 0.0)
    a = gamma.astype(f32) * lax.rsqrt(var + eps)
    bshift = beta.astype(f32) - a * mean
    a = a.reshape(Cout, 1)
    bshift = bshift.reshape(Cout, 1)

    kern2 = functools.partial(_bn_relu_kernel, H=H, W=W, Wp=Wp)
    out_flat = pl.pallas_call(
        kern2,
        grid=(N,),
        in_specs=[
            pl.BlockSpec((1, Cout, L2), lambda g: (g, 0, 0)),
            pl.BlockSpec((Cout, 1), lambda g: (0, 0)),
            pl.BlockSpec((Cout, 1), lambda g: (0, 0)),
        ],
        out_specs=pl.BlockSpec((1, Cout, HW), lambda g: (g, 0, 0)),
        out_shape=jax.ShapeDtypeStruct((N, Cout, HW), f32),
        compiler_params=cparams,
    )(z, a, bshift)

    x_out = out_flat.reshape(N, Cout, H, W)
    m_out = mo_flat.reshape(N, 1, H, W)
    return x_out, m_out
